# baseline (speedup 1.0000x reference)
import jax
import jax.numpy as jnp
from jax import lax
from jax.experimental import pallas as pl
from jax.experimental.pallas import tpu as pltpu

_N = 1024
_K = 16
_DIM = 64
_HID = 128
_ES = 128


def _min2(w):
    inf = jnp.inf
    m = w.reshape(_N // 8, 8, _ES)
    half = m.shape[0] // 2
    ma, mb = m[:half], m[half:]
    s = jnp.where(ma == mb, inf, jnp.maximum(ma, mb))
    m = jnp.minimum(ma, mb)
    while m.shape[0] > 1:
        half = m.shape[0] // 2
        ma, mb = m[:half], m[half:]
        sa, sb = s[:half], s[half:]
        s = jnp.minimum(jnp.minimum(sa, sb),
                        jnp.where(ma == mb, inf, jnp.maximum(ma, mb)))
        m = jnp.minimum(ma, mb)
    m, s = m[0], s[0]
    while m.shape[0] > 1:
        half = m.shape[0] // 2
        ma, mb = m[:half], m[half:]
        sa, sb = s[:half], s[half:]
        s = jnp.minimum(jnp.minimum(sa, sb),
                        jnp.where(ma == mb, inf, jnp.maximum(ma, mb)))
        m = jnp.minimum(ma, mb)
    return m, s


def _mask_body(xs_ref, xf_ref, h_ref, work_ref):
    xf = xf_ref[0, 0]
    xsub = xs_ref[0, 0]
    sq = jnp.sum(xf * xf, axis=1, keepdims=True)
    g = lax.dot_general(xf, xsub, (((1,), (1,)), ((), ())))
    score = sq - 2.0 * g

    _, s0 = _min2(score)
    work_ref[...] = jnp.where(score <= s0, jnp.inf, score)

    def pop2(_, carry):
        w = work_ref[...]
        _, s = _min2(w)
        work_ref[...] = jnp.where(w <= s, jnp.inf, w)
        return carry

    lax.fori_loop(0, _K // 2 - 2, pop2, 0, unroll=True)
    w = work_ref[...]
    _, m16 = _min2(w)
    h_ref[0, 0] = ((w <= m16) | jnp.isinf(w)).astype(jnp.bfloat16)


def _conv_body(h_ref, x_ref, w_ref, b_ref, o_ref):
    f32 = jnp.float32
    hh = h_ref[0, 0]
    x = x_ref[0, 0]
    W = w_ref[0]
    b = b_ref[0]

    ones_nh = jnp.ones((_N, _HID), jnp.bfloat16)
    ones_8n = jnp.ones((8, _N), jnp.bfloat16)

    dv = jnp.dot(hh, ones_nh, preferred_element_type=f32)[:, :1]
    inv_sqrt_dv = jnp.where(dv > 0, lax.rsqrt(dv), 0.0)
    isd_row = inv_sqrt_dv.T
    de_row = jnp.dot(ones_8n, hh, preferred_element_type=f32)[:1]

    y0t = lax.dot_general(W, x, (((0,), (1,)), ((), ()))) + b
    scaled_t = (y0t * isd_row).astype(jnp.bfloat16)
    zt = jnp.dot(scaled_t, hh, preferred_element_type=f32)
    zt = zt * (1.0 / de_row)
    z = zt.T.astype(jnp.bfloat16)
    out = jnp.dot(hh, z, preferred_element_type=f32) * inv_sqrt_dv
    out = jnp.where(out >= 0.0, out, 0.01 * out)
    s = lax.dot_general(ones_8n.astype(f32), out, (((1,), (0,)), ((), ())))
    o_ref[0, 0] = s[:1] * (1.0 / _N)


def kernel(x0, x1, c0, c1, W0, b0, W1, b1):
    del c0, c1
    xs = jnp.stack([x0, x1])
    ws = jnp.stack([W0, W1])
    bs = jnp.stack([b0, b1])[:, :, None]
    nb = x0.shape[0]

    incidence = pl.pallas_call(
        _mask_body,
        grid=(2, nb, _N // _ES),
        in_specs=[
            pl.BlockSpec((1, 1, _ES, _DIM), lambda l, b, s: (l, b, s, 0)),
            pl.BlockSpec((1, 1, _N, _DIM), lambda l, b, s: (l, b, 0, 0)),
        ],
        out_specs=pl.BlockSpec((1, 1, _N, _ES), lambda l, b, s: (l, b, 0, s)),
        out_shape=jax.ShapeDtypeStruct((2, nb, _N, _N), jnp.bfloat16),
        scratch_shapes=[pltpu.VMEM((_N, _ES), jnp.float32)],
    )(xs, xs)

    feats = pl.pallas_call(
        _conv_body,
        grid=(2, nb),
        in_specs=[
            pl.BlockSpec((1, 1, _N, _N), lambda l, b: (l, b, 0, 0)),
            pl.BlockSpec((1, 1, _N, _DIM), lambda l, b: (l, b, 0, 0)),
            pl.BlockSpec((1, _DIM, _HID), lambda l, b: (l, 0, 0)),
            pl.BlockSpec((1, _HID, 1), lambda l, b: (l, 0, 0)),
        ],
        out_specs=pl.BlockSpec((1, 1, 1, _HID), lambda l, b: (l, b, 0, 0)),
        out_shape=jax.ShapeDtypeStruct((2, nb, 1, _HID), jnp.float32),
    )(incidence, xs, ws, bs)

    return jnp.concatenate([feats[0, :, 0], feats[1, :, 0]], axis=-1)


# speedup vs baseline: 1.1078x; 1.1078x over previous
import jax
import jax.numpy as jnp
from jax import lax
from jax.experimental import pallas as pl
from jax.experimental.pallas import tpu as pltpu

_N = 1024
_K = 16
_DIM = 64
_HID = 128
_ES = 256


def _min2(w):
    inf = jnp.inf
    m = w.reshape(_N // 8, 8, _ES)
    half = m.shape[0] // 2
    ma, mb = m[:half], m[half:]
    s = jnp.where(ma == mb, inf, jnp.maximum(ma, mb))
    m = jnp.minimum(ma, mb)
    while m.shape[0] > 1:
        half = m.shape[0] // 2
        ma, mb = m[:half], m[half:]
        sa, sb = s[:half], s[half:]
        s = jnp.minimum(jnp.minimum(sa, sb),
                        jnp.where(ma == mb, inf, jnp.maximum(ma, mb)))
        m = jnp.minimum(ma, mb)
    m, s = m[0], s[0]
    while m.shape[0] > 1:
        half = m.shape[0] // 2
        ma, mb = m[:half], m[half:]
        sa, sb = s[:half], s[half:]
        s = jnp.minimum(jnp.minimum(sa, sb),
                        jnp.where(ma == mb, inf, jnp.maximum(ma, mb)))
        m = jnp.minimum(ma, mb)
    return m, s


def _mask_body(xs_ref, xf_ref, h_ref, work_ref):
    xf = xf_ref[0, 0]
    xsub = xs_ref[0, 0]
    sq = jnp.sum(xf * xf, axis=1, keepdims=True)
    g = lax.dot_general(xf, xsub, (((1,), (1,)), ((), ())))
    work_ref[...] = sq - 2.0 * g

    def pop2(_, carry):
        w = work_ref[...]
        _, s = _min2(w)
        work_ref[...] = jnp.where(w <= s, jnp.inf, w)
        return carry

    lax.fori_loop(0, _K // 2 - 1, pop2, 0, unroll=True)
    w = work_ref[...]
    _, m16 = _min2(w)
    h_ref[0, 0] = ((w <= m16) | jnp.isinf(w)).astype(jnp.bfloat16)


def _conv_body(h_ref, x_ref, w_ref, b_ref, o_ref):
    f32 = jnp.float32
    hh = h_ref[0, 0]
    x = x_ref[0, 0]
    W = w_ref[0]
    b = b_ref[0]

    ones_nh = jnp.ones((_N, _HID), jnp.bfloat16)
    ones_8n = jnp.ones((8, _N), jnp.bfloat16)

    dv = jnp.dot(hh, ones_nh, preferred_element_type=f32)[:, :1]
    inv_sqrt_dv = jnp.where(dv > 0, lax.rsqrt(dv), 0.0)
    isd_row = inv_sqrt_dv.T
    de_row = jnp.dot(ones_8n, hh, preferred_element_type=f32)[:1]

    y0t = lax.dot_general(W, x, (((0,), (1,)), ((), ()))) + b
    scaled_t = (y0t * isd_row).astype(jnp.bfloat16)
    zt = jnp.dot(scaled_t, hh, preferred_element_type=f32)
    zt = zt * (1.0 / de_row)
    z = zt.T.astype(jnp.bfloat16)
    out = jnp.dot(hh, z, preferred_element_type=f32) * inv_sqrt_dv
    out = jnp.where(out >= 0.0, out, 0.01 * out)
    s = lax.dot_general(ones_8n.astype(f32), out, (((1,), (0,)), ((), ())))
    o_ref[0, 0] = s[:1] * (1.0 / _N)


def kernel(x0, x1, c0, c1, W0, b0, W1, b1):
    del c0, c1
    xs = jnp.stack([x0, x1])
    ws = jnp.stack([W0, W1])
    bs = jnp.stack([b0, b1])[:, :, None]
    nb = x0.shape[0]

    incidence = pl.pallas_call(
        _mask_body,
        grid=(2, nb, _N // _ES),
        in_specs=[
            pl.BlockSpec((1, 1, _ES, _DIM), lambda l, b, s: (l, b, s, 0)),
            pl.BlockSpec((1, 1, _N, _DIM), lambda l, b, s: (l, b, 0, 0)),
        ],
        out_specs=pl.BlockSpec((1, 1, _N, _ES), lambda l, b, s: (l, b, 0, s)),
        out_shape=jax.ShapeDtypeStruct((2, nb, _N, _N), jnp.bfloat16),
        scratch_shapes=[pltpu.VMEM((_N, _ES), jnp.float32)],
    )(xs, xs)

    feats = pl.pallas_call(
        _conv_body,
        grid=(2, nb),
        in_specs=[
            pl.BlockSpec((1, 1, _N, _N), lambda l, b: (l, b, 0, 0)),
            pl.BlockSpec((1, 1, _N, _DIM), lambda l, b: (l, b, 0, 0)),
            pl.BlockSpec((1, _DIM, _HID), lambda l, b: (l, 0, 0)),
            pl.BlockSpec((1, _HID, 1), lambda l, b: (l, 0, 0)),
        ],
        out_specs=pl.BlockSpec((1, 1, 1, _HID), lambda l, b: (l, b, 0, 0)),
        out_shape=jax.ShapeDtypeStruct((2, nb, 1, _HID), jnp.float32),
    )(incidence, xs, ws, bs)

    return jnp.concatenate([feats[0, :, 0], feats[1, :, 0]], axis=-1)
